# baseline (speedup 1.0000x reference)
import functools

import jax
import jax.numpy as jnp
from jax import lax
from jax.experimental import pallas as pl
from jax.experimental.pallas import tpu as pltpu
from jax.experimental.pallas import tpu_sc as plsc

EMB = 32
NROW = 16384
NCOL = 26
L = 16

NC = 2
NS = 16
NW = NC * NS
ROWS_W = NROW // NW
CR = 32
NCHUNK = ROWS_W // CR
CE = CR * NCOL
GLENS = (128, 128, 128, 128, 128, 128, 64)
assert sum(GLENS) == CE


def _prep_body(emb0_ref, emb1_ref, emb2_ref, w_ref, b_ref, t_ref):
    w = w_ref[...]
    dn = (((1,), (1,)), ((), ()))
    f0 = lax.dot_general(emb0_ref[...], w[:, 0:32], dn,
                         preferred_element_type=jnp.float32)
    f1 = lax.dot_general(emb1_ref[...], w[:, 32:64], dn,
                         preferred_element_type=jnp.float32)
    r2 = lax.dot_general(emb2_ref[0:1, :], w[:, 64:96], dn,
                         preferred_element_type=jnp.float32)
    t_ref[0:1024, :] = f0 + r2 + b_ref[...]
    t_ref[1024:2048, :] = f1


_prep = pl.pallas_call(
    _prep_body,
    out_shape=jax.ShapeDtypeStruct((2 * 1024, EMB), jnp.float32),
)


def _sc_body(x_hbm, tab_hbm, out_hbm,
             xv0, xv1, i0a, i1a, i0b, i1b, g0a, g1a, g0b, g1b,
             semga, semgb, semo):
    wid = lax.axis_index("s") * NC + lax.axis_index("c")
    row0 = wid * ROWS_W

    xv = (xv0, xv1)
    i0 = (i0a, i0b)
    i1 = (i1a, i1b)
    g0 = (g0a, g0b)
    g1 = (g1a, g1b)
    semg = (semga, semgb)

    def load_and_index(ci, s):
        pltpu.sync_copy(x_hbm.at[pl.ds(row0 + ci * CR, CR)], xv[s])
        i0s, i1s, xvs = i0[s], i1[s], xv[s]

        def idx_body(r, c):
            xa = xvs[r, pl.ds(0, L)]
            xb = xvs[r, pl.ds(NCOL - L, L)]
            base = r * NCOL
            i0s[pl.ds(base, L)] = lax.bitwise_and(xa, 1023)
            i0s[pl.ds(base + NCOL - L, L)] = lax.bitwise_and(xb, 1023)
            i1s[pl.ds(base, L)] = lax.bitwise_and(
                lax.shift_right_logical(xa, 10), 1023) + 1024
            i1s[pl.ds(base + NCOL - L, L)] = lax.bitwise_and(
                lax.shift_right_logical(xb, 10), 1023) + 1024
            return c

        lax.fori_loop(0, CR, idx_body, 0)

    def fire_gathers(s):
        descs = []
        off = 0
        for ln in GLENS:
            descs.append(pltpu.async_copy(
                tab_hbm.at[i0[s].at[pl.ds(off, ln)]],
                g0[s].at[pl.ds(off, ln)], semg[s]))
            descs.append(pltpu.async_copy(
                tab_hbm.at[i1[s].at[pl.ds(off, ln)]],
                g1[s].at[pl.ds(off, ln)], semg[s]))
            off += ln
        return descs

    def accumulate(s):
        g0s, g1s = g0[s], g1[s]

        def add_body(r4, c):
            r = r4 * 4
            for u in range(4):
                for h in range(EMB // L):
                    v = g1s[r + u, pl.ds(h * L, L)]
                    plsc.addupdate(g0s.at[r + u, pl.ds(h * L, L)], v)
            return c

        lax.fori_loop(0, CE // 4, add_body, 0)

    def fire_out(ci, s):
        rb = row0 + ci * CR
        g0s = g0[s]

        def out_body(r, c):
            pltpu.async_copy(g0s.at[pl.ds(r * NCOL, NCOL)],
                             out_hbm.at[rb + r], semo)
            return c

        lax.fori_loop(0, CR, out_body, 0)

    def drain_out(ci, s):
        rb = row0 + ci * CR
        g0s = g0[s]

        def drain_body(r, c):
            pltpu.make_async_copy(g0s.at[pl.ds(r * NCOL, NCOL)],
                                  out_hbm.at[rb + r], semo).wait()
            return c

        lax.fori_loop(0, CR, drain_body, 0)

    load_and_index(0, 0)
    descs_cur = fire_gathers(0)
    for i in range(NCHUNK):
        p, q = i % 2, (i + 1) % 2
        descs_next = None
        if i + 1 < NCHUNK:
            load_and_index(i + 1, q)
            if i >= 1:
                drain_out(i - 1, q)
            descs_next = fire_gathers(q)
        for d in descs_cur:
            d.wait()
        accumulate(p)
        fire_out(i, p)
        descs_cur = descs_next
    drain_out(NCHUNK - 2, (NCHUNK - 2) % 2)
    drain_out(NCHUNK - 1, (NCHUNK - 1) % 2)


_sc_gather = functools.partial(
    pl.kernel,
    out_type=jax.ShapeDtypeStruct((NROW, NCOL, EMB), jnp.float32),
    mesh=plsc.VectorSubcoreMesh(core_axis_name="c", subcore_axis_name="s",
                                num_cores=NC, num_subcores=NS),
    scratch_types=[
        pltpu.VMEM((CR, NCOL), jnp.int32),
        pltpu.VMEM((CR, NCOL), jnp.int32),
        pltpu.VMEM((CE,), jnp.int32),
        pltpu.VMEM((CE,), jnp.int32),
        pltpu.VMEM((CE,), jnp.int32),
        pltpu.VMEM((CE,), jnp.int32),
        pltpu.VMEM((CE, EMB), jnp.float32),
        pltpu.VMEM((CE, EMB), jnp.float32),
        pltpu.VMEM((CE, EMB), jnp.float32),
        pltpu.VMEM((CE, EMB), jnp.float32),
        pltpu.SemaphoreType.DMA,
        pltpu.SemaphoreType.DMA,
        pltpu.SemaphoreType.DMA,
    ],
    compiler_params=pltpu.CompilerParams(use_tc_tiling_on_sc=False),
)(_sc_body)


def kernel(x, emb0, emb1, emb2, W, b):
    tab = _prep(emb0, emb1, emb2, W, b.reshape(1, EMB))
    return _sc_gather(x, tab)


# speedup vs baseline: 1.0107x; 1.0107x over previous
import functools

import jax
import jax.numpy as jnp
from jax import lax
from jax.experimental import pallas as pl
from jax.experimental.pallas import tpu as pltpu
from jax.experimental.pallas import tpu_sc as plsc

EMB = 32
NROW = 16384
NCOL = 26
L = 16

NC = 2
NS = 16
NW = NC * NS
ROWS_W = NROW // NW
HROWS = ROWS_W // 2
NGRP = HROWS // L
TPAD = 33


def _prep_body(emb0_ref, emb1_ref, emb2_ref, w_ref, b_ref, t_ref):
    w = w_ref[...]
    dn = (((1,), (1,)), ((), ()))
    f0 = lax.dot_general(emb0_ref[...], w[:, 0:32], dn,
                         preferred_element_type=jnp.float32)
    f1 = lax.dot_general(emb1_ref[...], w[:, 32:64], dn,
                         preferred_element_type=jnp.float32)
    r2 = lax.dot_general(emb2_ref[0:1, :], w[:, 64:96], dn,
                         preferred_element_type=jnp.float32)
    t_ref[0:1024, :] = f0 + r2 + b_ref[...]
    t_ref[1024:2048, :] = f1


_prep = pl.pallas_call(
    _prep_body,
    out_shape=jax.ShapeDtypeStruct((2 * 1024, EMB), jnp.float32),
)


def _sc_body(x_hbm, tab_hbm, p_hbm,
             xv, tabv, tmp, stag0, stag1, semd0, semd1):
    wid = lax.axis_index("s") * NC + lax.axis_index("c")
    i0 = wid * ROWS_W

    pltpu.sync_copy(x_hbm.at[pl.ds(i0, ROWS_W)], xv)

    def stage_tab(t, carry):
        pltpu.sync_copy(tab_hbm.at[pl.ds(t * 256, 256)], tmp)

        def expand_body(r, c):
            for h in range(EMB // L):
                tabv[t * 256 + r, pl.ds(h * L, L)] = tmp[r, pl.ds(h * L, L)]
            return c

        lax.fori_loop(0, 256, expand_body, 0)
        return carry

    lax.fori_loop(0, 8, stage_tab, 0)

    stag = (stag0, stag1)
    semd = (semd0, semd1)
    iota = lax.broadcasted_iota(jnp.int32, (L,), 0)

    def col_body(j, carry):
        jv = jnp.full((L,), 0, jnp.int32) + j
        for t in range(2):
            @pl.when(j >= 1)
            def _wait():
                pltpu.make_async_copy(
                    stag[t], p_hbm.at[0, :, pl.ds(i0, HROWS)],
                    semd[t]).wait()

            def grp_body(ii, c, _t=t, _jv=jv):
                iv = (_t * NGRP + ii) * L + iota
                xw = plsc.load_gather(xv, [iv, _jv])
                c0 = lax.bitwise_and(xw, 1023)
                c1 = lax.bitwise_and(lax.shift_right_logical(xw, 10),
                                     1023) + 1024
                for k in range(EMB):
                    kv = jnp.full((L,), k, jnp.int32)
                    v0 = plsc.load_gather(tabv, [c0, kv])
                    v1 = plsc.load_gather(tabv, [c1, kv])
                    stag[_t][k, pl.ds(ii * L, L)] = v0 + v1
                return c

            lax.fori_loop(0, NGRP, grp_body, 0)
            pltpu.async_copy(
                stag[t], p_hbm.at[j, :, pl.ds(i0 + t * HROWS, HROWS)],
                semd[t])
        return carry

    lax.fori_loop(0, NCOL, col_body, 0)
    for t in range(2):
        pltpu.make_async_copy(stag[t], p_hbm.at[0, :, pl.ds(i0, HROWS)],
                              semd[t]).wait()


_sc_t = functools.partial(
    pl.kernel,
    out_type=jax.ShapeDtypeStruct((NCOL, EMB, NROW), jnp.float32),
    mesh=plsc.VectorSubcoreMesh(core_axis_name="c", subcore_axis_name="s",
                                num_cores=NC, num_subcores=NS),
    scratch_types=[
        pltpu.VMEM((ROWS_W, NCOL), jnp.int32),
        pltpu.VMEM((2 * 1024, TPAD), jnp.float32),
        pltpu.VMEM((256, EMB), jnp.float32),
        pltpu.VMEM((EMB, HROWS), jnp.float32),
        pltpu.VMEM((EMB, HROWS), jnp.float32),
        pltpu.SemaphoreType.DMA,
        pltpu.SemaphoreType.DMA,
    ],
    compiler_params=pltpu.CompilerParams(use_tc_tiling_on_sc=False,
                                         needs_layout_passes=False),
)(_sc_body)


def kernel(x, emb0, emb1, emb2, W, b):
    tab = _prep(emb0, emb1, emb2, W, b.reshape(1, EMB))
    p = _sc_t(x, tab)
    return jnp.transpose(p, (2, 0, 1))


# speedup vs baseline: 1.9221x; 1.9017x over previous
import functools

import jax
import jax.numpy as jnp
from jax import lax
from jax.experimental import pallas as pl
from jax.experimental.pallas import tpu as pltpu
from jax.experimental.pallas import tpu_sc as plsc

EMB = 32
NROW = 16384
NCOL = 26
L = 16

NC = 2
NS = 16
NW = NC * NS
ROWS_W = NROW // NW
HROWS = ROWS_W // 2
SPAD = HROWS + 1
UNROLL = 4


def _prep_body(emb0_ref, emb1_ref, emb2_ref, w_ref, b_ref, t_ref):
    w = w_ref[...]
    dn = (((1,), (1,)), ((), ()))
    f0 = lax.dot_general(emb0_ref[...], w[:, 0:32], dn,
                         preferred_element_type=jnp.float32)
    f1 = lax.dot_general(emb1_ref[...], w[:, 32:64], dn,
                         preferred_element_type=jnp.float32)
    r2 = lax.dot_general(emb2_ref[0:1, :], w[:, 64:96], dn,
                         preferred_element_type=jnp.float32)
    t_ref[0:1024, :] = f0 + r2 + b_ref[...]
    t_ref[1024:2048, :] = f1


_prep = pl.pallas_call(
    _prep_body,
    out_shape=jax.ShapeDtypeStruct((2 * 1024, EMB), jnp.float32),
)


def _sc_body(x_hbm, tab_hbm, p_hbm,
             xv, tabv, stag0, stag1, semd0, semd1):
    wid = lax.axis_index("s") * NC + lax.axis_index("c")
    i0 = wid * ROWS_W

    pltpu.sync_copy(x_hbm.at[pl.ds(i0, ROWS_W)], xv)
    pltpu.sync_copy(tab_hbm, tabv)

    stag = (stag0, stag1)
    semd = (semd0, semd1)
    klo = lax.broadcasted_iota(jnp.int32, (L,), 0)
    khi = klo + L
    zv = jnp.full((L,), 0, jnp.int32)

    def col_body(j, carry):
        jv = zv + j
        for t in range(2):
            @pl.when(j >= 1)
            def _wait():
                pltpu.make_async_copy(
                    stag[t].at[:, pl.ds(0, HROWS)],
                    p_hbm.at[0, :, pl.ds(i0, HROWS)],
                    semd[t]).wait()

            def elem_body(ii, c, _t=t, _jv=jv):
                ib = ii * L
                iv = zv + ib + klo
                xw = plsc.load_gather(xv, [iv + _t * HROWS, _jv])
                c0v = lax.bitwise_and(xw, 1023)
                c1v = lax.bitwise_and(
                    lax.shift_right_logical(xw, 10), 1023) + 1024
                for u in range(L):
                    c0 = c0v[u]
                    c1 = c1v[u]
                    a = tabv[c0, pl.ds(0, L)] + tabv[c1, pl.ds(0, L)]
                    bb = tabv[c0, pl.ds(L, L)] + tabv[c1, pl.ds(L, L)]
                    uv = zv + ib + u
                    plsc.store_scatter(stag[_t], [klo, uv], a)
                    plsc.store_scatter(stag[_t], [khi, uv], bb)
                return c

            lax.fori_loop(0, HROWS // L, elem_body, 0)
            pltpu.async_copy(
                stag[t].at[:, pl.ds(0, HROWS)],
                p_hbm.at[j, :, pl.ds(i0 + t * HROWS, HROWS)],
                semd[t])
        return carry

    lax.fori_loop(0, NCOL, col_body, 0)
    for t in range(2):
        pltpu.make_async_copy(stag[t].at[:, pl.ds(0, HROWS)],
                              p_hbm.at[0, :, pl.ds(i0, HROWS)],
                              semd[t]).wait()


_sc_t = functools.partial(
    pl.kernel,
    out_type=jax.ShapeDtypeStruct((NCOL, EMB, NROW), jnp.float32),
    mesh=plsc.VectorSubcoreMesh(core_axis_name="c", subcore_axis_name="s",
                                num_cores=NC, num_subcores=NS),
    scratch_types=[
        pltpu.VMEM((ROWS_W, NCOL), jnp.int32),
        pltpu.VMEM((2 * 1024, EMB), jnp.float32),
        pltpu.VMEM((EMB, SPAD), jnp.float32),
        pltpu.VMEM((EMB, SPAD), jnp.float32),
        pltpu.SemaphoreType.DMA,
        pltpu.SemaphoreType.DMA,
    ],
    compiler_params=pltpu.CompilerParams(use_tc_tiling_on_sc=False,
                                         needs_layout_passes=False),
)(_sc_body)


def kernel(x, emb0, emb1, emb2, W, b):
    tab = _prep(emb0, emb1, emb2, W, b.reshape(1, EMB))
    p = _sc_t(x, tab)
    return jnp.transpose(p, (2, 0, 1))


# speedup vs baseline: 2.5827x; 1.3437x over previous
import functools

import jax
import jax.numpy as jnp
from jax import lax
from jax.experimental import pallas as pl
from jax.experimental.pallas import tpu as pltpu
from jax.experimental.pallas import tpu_sc as plsc

EMB = 32
NROW = 16384
NCOL = 26
L = 16

NC = 2
NS = 16
NW = NC * NS
ROWS_W = NROW // NW
HROWS = ROWS_W // 2
SPAD = HROWS + 1
UNROLL = 4


def _prep_body(emb0_ref, emb1_ref, emb2_ref, w_ref, b_ref, t_ref):
    w = w_ref[...]
    dn = (((1,), (1,)), ((), ()))
    f0 = lax.dot_general(emb0_ref[...], w[:, 0:32], dn,
                         preferred_element_type=jnp.float32)
    f1 = lax.dot_general(emb1_ref[...], w[:, 32:64], dn,
                         preferred_element_type=jnp.float32)
    r2 = lax.dot_general(emb2_ref[0:1, :], w[:, 64:96], dn,
                         preferred_element_type=jnp.float32)
    t_ref[0:1024, :] = f0 + r2 + b_ref[...]
    t_ref[1024:2048, :] = f1


_prep = pl.pallas_call(
    _prep_body,
    out_shape=jax.ShapeDtypeStruct((2 * 1024, EMB), jnp.float32),
)


def _sc_body(x_hbm, tab_hbm, p_hbm,
             xv, tabv, stag0, stag1, semd0, semd1):
    wid = lax.axis_index("s") * NC + lax.axis_index("c")
    i0 = wid * ROWS_W

    pltpu.sync_copy(x_hbm.at[pl.ds(i0, ROWS_W)], xv)
    pltpu.sync_copy(tab_hbm, tabv)

    stag = (stag0, stag1)
    semd = (semd0, semd1)
    klo = lax.broadcasted_iota(jnp.int32, (L,), 0)
    khi = klo + L
    zv = jnp.full((L,), 0, jnp.int32)

    def col_body(j, carry):
        jv = zv + j
        for t in range(2):
            @pl.when(j >= 1)
            def _wait():
                for _ in range(8):
                    pltpu.make_async_copy(
                        stag[t].at[pl.ds(0, 8), pl.ds(0, 128)],
                        p_hbm.at[0, 0, 0, :, :],
                        semd[t]).wait()

            def elem_body(ii, c, _t=t, _jv=jv):
                ib = ii * L
                iv = zv + ib + klo
                xw = plsc.load_gather(xv, [iv + _t * HROWS, _jv])
                c0v = lax.bitwise_and(xw, 1023)
                c1v = lax.bitwise_and(
                    lax.shift_right_logical(xw, 10), 1023) + 1024
                for u in range(L):
                    c0 = c0v[u]
                    c1 = c1v[u]
                    a = tabv[c0, pl.ds(0, L)] + tabv[c1, pl.ds(0, L)]
                    bb = tabv[c0, pl.ds(L, L)] + tabv[c1, pl.ds(L, L)]
                    uv = zv + ib + u
                    plsc.store_scatter(stag[_t], [klo, uv], a)
                    plsc.store_scatter(stag[_t], [khi, uv], bb)
                return c

            lax.fori_loop(0, HROWS // L, elem_body, 0)
            ti0 = (i0 + t * HROWS) // 128
            for tk in range(4):
                for tl in range(2):
                    pltpu.async_copy(
                        stag[t].at[pl.ds(tk * 8, 8), pl.ds(tl * 128, 128)],
                        p_hbm.at[j, tk, ti0 + tl, :, :],
                        semd[t])
        return carry

    lax.fori_loop(0, NCOL, col_body, 0)
    for t in range(2):
        for _ in range(8):
            pltpu.make_async_copy(stag[t].at[pl.ds(0, 8), pl.ds(0, 128)],
                                  p_hbm.at[0, 0, 0, :, :],
                                  semd[t]).wait()


_sc_t = functools.partial(
    pl.kernel,
    out_type=jax.ShapeDtypeStruct((NCOL, 4, NROW // 128, 8, 128),
                                  jnp.float32),
    mesh=plsc.VectorSubcoreMesh(core_axis_name="c", subcore_axis_name="s",
                                num_cores=NC, num_subcores=NS),
    scratch_types=[
        pltpu.VMEM((ROWS_W, NCOL), jnp.int32),
        pltpu.VMEM((2 * 1024, EMB), jnp.float32),
        pltpu.VMEM((EMB, SPAD), jnp.float32),
        pltpu.VMEM((EMB, SPAD), jnp.float32),
        pltpu.SemaphoreType.DMA,
        pltpu.SemaphoreType.DMA,
    ],
    compiler_params=pltpu.CompilerParams(use_tc_tiling_on_sc=False,
                                         needs_layout_passes=False),
)(_sc_body)


def kernel(x, emb0, emb1, emb2, W, b):
    tab = _prep(emb0, emb1, emb2, W, b.reshape(1, EMB))
    p = _sc_t(x, tab)
    return jnp.transpose(p, (2, 4, 0, 1, 3)).reshape(NROW, NCOL, EMB)
